# baseline (speedup 1.0000x reference)
import functools

import jax
import jax.numpy as jnp
from jax import lax
from jax.experimental import pallas as pl
from jax.experimental.pallas import tpu as pltpu
from jax.experimental.pallas import tpu_sc as plsc

VOCAB = 1000000
HIDDEN = 64
BATCH = 16384
FIELDS = 26

_NW = 32
_BW = BATCH // _NW
_CH = 128
_NSUB = _BW // _CH
_NSTEP = FIELDS * _NSUB
_PER_W = FIELDS * _BW


@functools.partial(
    pl.kernel,
    mesh=plsc.VectorSubcoreMesh(core_axis_name="c", subcore_axis_name="s"),
    out_type=jax.ShapeDtypeStruct((FIELDS, HIDDEN, BATCH), jnp.float32),
    scratch_types=[
        pltpu.VMEM((_PER_W,), jnp.int32),
        pltpu.VMEM((_PER_W,), jnp.int32),
        pltpu.VMEM((_PER_W,), jnp.int32),
        pltpu.VMEM((2, _CH, 2 * HIDDEN), jnp.float32),
        pltpu.VMEM((2, HIDDEN, _CH), jnp.float32),
        pltpu.SemaphoreType.DMA,
        pltpu.SemaphoreType.DMA((2,)),
        pltpu.SemaphoreType.DMA((2,)),
    ],
    compiler_params=pltpu.CompilerParams(needs_layout_passes=False),
)
def _emb_lookup(xlin_hbm, tablep_hbm, out_hbm, idx_all, p_all, h2_all,
                rows_v, cols_v, i_sem, g_sem, o_sem):
    wid = lax.axis_index("s") * 2 + lax.axis_index("c")
    base_b = wid * _BW

    descs = []
    for f in range(FIELDS):
        descs.append(pltpu.async_copy(
            xlin_hbm.at[pl.ds(f * BATCH + base_b, _BW)],
            idx_all.at[pl.ds(f * _BW, _BW)],
            i_sem,
        ))
    for d in descs:
        d.wait()

    def prep(i, carry):
        v = idx_all[pl.ds(i * 16, 16)]
        p_all[pl.ds(i * 16, 16)] = lax.shift_right_logical(v, 1)
        h2_all[pl.ds(i * 16, 16)] = (v & 1) * HIDDEN
        return carry

    lax.fori_loop(0, _PER_W // 16, prep, 0)

    def gstart(s, b):
        return pltpu.async_copy(
            tablep_hbm.at[p_all.at[pl.ds(s * _CH, _CH)]],
            rows_v.at[b],
            g_sem.at[b],
        )

    def gwait(s, b):
        pltpu.make_async_copy(
            tablep_hbm.at[p_all.at[pl.ds(s * _CH, _CH)]],
            rows_v.at[b],
            g_sem.at[b],
        ).wait()

    def owait(s, b):
        f = s // _NSUB
        sub = lax.rem(s, _NSUB)
        pltpu.make_async_copy(
            cols_v.at[b],
            out_hbm.at[f, :, pl.ds(base_b + sub * _CH, _CH)],
            o_sem.at[b],
        ).wait()

    def shuffle_and_write(s, b):
        for cb in range(_CH // 16):
            cvec = lax.iota(jnp.int32, 16) + cb * 16
            colbase = h2_all[pl.ds(s * _CH + cb * 16, 16)]

            for h in range(HIDDEN):
                vals = plsc.load_gather(rows_v.at[b], [cvec, colbase + h])
                cols_v[b, h, pl.ds(cb * 16, 16)] = vals
        f = s // _NSUB
        sub = lax.rem(s, _NSUB)
        pltpu.async_copy(
            cols_v.at[b],
            out_hbm.at[f, :, pl.ds(base_b + sub * _CH, _CH)],
            o_sem.at[b],
        )

    gstart(0, 0)

    def body(s, carry):
        b = lax.rem(s, 2)

        @pl.when(s + 1 < _NSTEP)
        def _():
            gstart(s + 1, 1 - b)

        gwait(s, b)

        @pl.when(s >= 2)
        def _():
            owait(s - 2, b)

        shuffle_and_write(s, b)
        return carry

    lax.fori_loop(0, _NSTEP, body, 0)
    owait(_NSTEP - 2, 0)
    owait(_NSTEP - 1, 1)


def kernel(x, embedding):
    xlin = x.T.reshape(-1)
    tablep = embedding.reshape(VOCAB // 2, 2 * HIDDEN)
    out_phys = _emb_lookup(xlin, tablep)
    return out_phys.transpose(2, 0, 1)


# speedup vs baseline: 1.2667x; 1.2667x over previous
import functools

import jax
import jax.numpy as jnp
from jax import lax
from jax.experimental import pallas as pl
from jax.experimental.pallas import tpu as pltpu
from jax.experimental.pallas import tpu_sc as plsc

VOCAB = 1000000
HIDDEN = 64
BATCH = 16384
FIELDS = 26

_NW = 32
_BW = BATCH // _NW
_CH = 128
_NSUB = _BW // _CH
_NSTEP = FIELDS * _NSUB
_PER_W = FIELDS * _BW


@functools.partial(
    pl.kernel,
    mesh=plsc.VectorSubcoreMesh(core_axis_name="c", subcore_axis_name="s"),
    out_type=jax.ShapeDtypeStruct((FIELDS, HIDDEN, BATCH), jnp.float32),
    scratch_types=[
        pltpu.VMEM((_PER_W,), jnp.int32),
        pltpu.VMEM((_PER_W,), jnp.int32),
        pltpu.VMEM((_PER_W,), jnp.int32),
        pltpu.VMEM((2, _CH, 2 * HIDDEN), jnp.float32),
        pltpu.VMEM((2, HIDDEN, _CH), jnp.float32),
        pltpu.SemaphoreType.DMA,
        pltpu.SemaphoreType.DMA((2,)),
        pltpu.SemaphoreType.DMA((2,)),
    ],
    compiler_params=pltpu.CompilerParams(needs_layout_passes=False),
)
def _emb_lookup(xlin_hbm, tablep_hbm, out_hbm, idx_all, p_all, h2_all,
                rows_v, cols_v, i_sem, g_sem, o_sem):
    wid = lax.axis_index("s") * 2 + lax.axis_index("c")
    base_b = wid * _BW

    descs = []
    for f in range(FIELDS):
        descs.append(pltpu.async_copy(
            xlin_hbm.at[pl.ds(f * BATCH + base_b, _BW)],
            idx_all.at[pl.ds(f * _BW, _BW)],
            i_sem,
        ))
    for d in descs:
        d.wait()

    def prep(i, carry):
        v = idx_all[pl.ds(i * 16, 16)]
        p_all[pl.ds(i * 16, 16)] = lax.shift_right_logical(v, 1)
        h2_all[pl.ds(i * 16, 16)] = (v & 1) * HIDDEN
        return carry

    lax.fori_loop(0, _PER_W // 16, prep, 0)

    def gstart(s, b):
        return pltpu.async_copy(
            tablep_hbm.at[p_all.at[pl.ds(s * _CH, _CH)]],
            rows_v.at[b],
            g_sem.at[b],
        )

    def gwait(s, b):
        pltpu.make_async_copy(
            tablep_hbm.at[p_all.at[pl.ds(s * _CH, _CH)]],
            rows_v.at[b],
            g_sem.at[b],
        ).wait()

    def owait(s, b):
        f = s // _NSUB
        sub = lax.rem(s, _NSUB)
        pltpu.make_async_copy(
            cols_v.at[b],
            out_hbm.at[f, :, pl.ds(base_b + sub * _CH, _CH)],
            o_sem.at[b],
        ).wait()

    def shuffle_and_write(s, b):
        iot = lax.iota(jnp.int32, 16)

        @plsc.parallel_loop(0, HIDDEN, unroll=4)
        def _(h):
            for cb in range(_CH // 16):
                cvec = iot + cb * 16
                colbase = h2_all[pl.ds(s * _CH + cb * 16, 16)]
                vals = plsc.load_gather(rows_v.at[b], [cvec, colbase + h])
                cols_v[b, h, pl.ds(cb * 16, 16)] = vals
        f = s // _NSUB
        sub = lax.rem(s, _NSUB)
        pltpu.async_copy(
            cols_v.at[b],
            out_hbm.at[f, :, pl.ds(base_b + sub * _CH, _CH)],
            o_sem.at[b],
        )

    gstart(0, 0)

    def body(s, carry):
        b = lax.rem(s, 2)

        @pl.when(s + 1 < _NSTEP)
        def _():
            gstart(s + 1, 1 - b)

        gwait(s, b)

        @pl.when(s >= 2)
        def _():
            owait(s - 2, b)

        shuffle_and_write(s, b)
        return carry

    lax.fori_loop(0, _NSTEP, body, 0)
    owait(_NSTEP - 2, 0)
    owait(_NSTEP - 1, 1)


def kernel(x, embedding):
    xlin = x.T.reshape(-1)
    tablep = embedding.reshape(VOCAB // 2, 2 * HIDDEN)
    out_phys = _emb_lookup(xlin, tablep)
    return out_phys.transpose(2, 0, 1)
